# baseline (speedup 1.0000x reference)
import functools

import jax
import jax.numpy as jnp
from jax.experimental import pallas as pl
from jax.experimental.pallas import tpu as pltpu
from jax.experimental.pallas import tpu_sc as plsc

F32 = jnp.float32
NEG = -1e30
B = 4
NS = 128
F = 128
H = 4
HE = 64
KPOOL = (64, 32)
NREAL = 98
NP = 128
KEDGE = 10
LAST = 97
NT = 104

_PREC = jax.lax.Precision.DEFAULT


def _mm(a, b):
    return jax.lax.dot_general(a, b, (((1,), (0,)), ((), ())),
                               precision=_PREC, preferred_element_type=F32)


def _mmT(a, b):
    return jax.lax.dot_general(a, b, (((1,), (1,)), ((), ())),
                               precision=_PREC, preferred_element_type=F32)


def _sigmoid(x):
    return 1.0 / (1.0 + jnp.exp(-x))


def _elu(x):
    return jnp.where(x > 0, x, jnp.exp(jnp.minimum(x, 0.0)) - 1.0)


def _topk_mask(e, k):
    col = jax.lax.broadcasted_iota(jnp.int32, e.shape, e.ndim - 1)
    work = e
    mask = jnp.zeros_like(e)
    for _ in range(k):
        m = jnp.max(work, axis=-1, keepdims=True)
        cand = jnp.where(work == m, col, jnp.int32(1 << 20))
        first = jnp.min(cand, axis=-1, keepdims=True)
        sel = col == first
        mask = jnp.where(sel, 1.0, mask)
        work = jnp.where(sel, NEG, work)
    return mask


def _masked_softmax_att(eraw, colI):
    valid = colI < NREAL
    esel = jnp.where(valid, eraw, NEG)
    mask = _topk_mask(esel, KEDGE)
    e = jnp.where(valid, eraw * mask, NEG)
    m = jnp.max(e, axis=-1, keepdims=True)
    p = jnp.exp(e - m)
    return p / jnp.sum(p, axis=-1, keepdims=True)


def _scores_body(hs8, pw1, pb1, pw2, out):
    for s in range(2):
        w1 = pw1[s]
        b1 = pb1[s]
        w2 = pw2[s]
        for b in range(B):
            X = hs8[s * B + b]
            t1 = jnp.tanh(_mmT(X, w1) + b1)
            out[s * B + b:s * B + b + 1, :] = _mmT(w2, t1)


def _pool_sc_body(scores, hs8, out, sc_v, idx_v, rows_v, sem):
    nc = 2
    wid = jax.lax.axis_index("s") * nc + jax.lax.axis_index("c")

    @pl.when(wid < 8)
    def _():
        pltpu.sync_copy(scores.at[wid], sc_v)

        lane = jax.lax.iota(jnp.int32, 16)

        def _bfly(v, op):
            for k in (8, 4, 2, 1):
                v = op(v, v.at[lane ^ k].get(mode="promise_in_bounds"))
            return v

        chunks = [sc_v[pl.ds(16 * c, 16)] for c in range(8)]
        gids = [16 * c + lane for c in range(8)]
        big = jnp.full((16,), 1 << 20, jnp.int32)
        negv = jnp.full((16,), -3e38, F32)
        acc = jnp.zeros((16,), jnp.int32)
        for t in range(64):
            m = chunks[0]
            for c in range(1, 8):
                m = jnp.maximum(m, chunks[c])
            m = _bfly(m, jnp.maximum)
            cand = big
            for c in range(8):
                cand = jnp.minimum(cand,
                                   jnp.where(chunks[c] == m, gids[c], big))
            gidx = _bfly(cand, jnp.minimum)
            acc = jnp.where(lane == (t % 16), gidx, acc)
            if t % 16 == 15:
                idx_v[pl.ds(16 * (t // 16), 16)] = acc
                acc = jnp.zeros((16,), jnp.int32)
            for c in range(8):
                chunks[c] = jnp.where(gids[c] == gidx, negv, chunks[c])
        pltpu.async_copy(hs8.at[wid].at[idx_v], rows_v, sem).wait()
        pltpu.sync_copy(rows_v, out.at[wid])


def _pool_sc(scores, hs8):
    fn = functools.partial(
        pl.kernel,
        out_type=jax.ShapeDtypeStruct((8, 64, F), F32),
        mesh=plsc.VectorSubcoreMesh(core_axis_name="c", subcore_axis_name="s"),
        scratch_types=[
            pltpu.VMEM((NS,), F32),
            pltpu.VMEM((64,), jnp.int32),
            pltpu.VMEM((64, F), F32),
            pltpu.SemaphoreType.DMA,
        ],
    )(_pool_sc_body)
    return fn(scores, hs8)


def _body(pooled8, meta, cls, headW, Ai, Aj, Am, eb1c, ew2c, eb2,
          uw1, ub1r, uw2r, ub2, fcw, fcbr, lng, lnb, projw, projbr, out):
    meta_a = meta[...]
    cls_r = cls[...]

    zpad = jnp.zeros((NP - NREAL, F), dtype=F32)
    hrows = []
    for b in range(B):
        hrows.append(jnp.concatenate(
            [pooled8[b], pooled8[B + b][0:KPOOL[1]], meta_a[b:b + 1, :],
             cls_r, zpad],
            axis=0))
    hf = jnp.concatenate(hrows, axis=0)

    colI = jax.lax.broadcasted_iota(jnp.int32, (H * B * NT, NP), 1)
    colI2 = jax.lax.broadcasted_iota(jnp.int32, (H * B, NP), 1)

    l = 0
    res = hf
    Wh_l = []
    u_l = []
    E_l = []
    for hd in range(H):
        W = headW[l, hd]
        Wh = _mmT(hf, W)
        Wh_l.append(Wh)
        si = _mmT(Wh, Ai[l, hd])
        smT = _mmT(Am[l, hd], meta_a) + eb1c[l, hd]
        tu = jnp.maximum(_mmT(hf, uw1[l, hd]) + ub1r[l, hd], 0.0)
        ub = []
        for b in range(B):
            Whb = Wh[b * NP:(b + 1) * NP]
            TJt = _mmT(Aj[l, hd], Whb) + smT[:, b:b + 1]
            sib = si[b * NP:b * NP + NT]
            Hd = jnp.maximum(sib[:, :, None] + TJt[None, :, :], 0.0)
            Eb = jnp.sum(Hd * ew2c[l, hd][None, :, :], axis=1) + eb2[l, hd]
            E_l.append(Eb)
            utb = _sigmoid(_mmT(uw2r[l, hd], tu[b * NP:(b + 1) * NP])
                           + ub2[l, hd])
            ub.append(utb)
        u_l.append(ub)

    E_all = jnp.concatenate(E_l, axis=0)
    att_all = _masked_softmax_att(E_all, colI)
    hcat_b = [[] for _ in range(B)]
    for hd in range(H):
        for b in range(B):
            att = att_all[(hd * B + b) * NT:(hd * B + b + 1) * NT]
            att = att * u_l[hd][b]
            hcat_b[b].append(_mm(att, Wh_l[hd][b * NP:(b + 1) * NP]))
    hcat = jnp.concatenate(
        [jnp.concatenate(hcat_b[b], axis=1) for b in range(B)], axis=0)
    res_t = jnp.concatenate(
        [hf[b * NP:b * NP + NT] for b in range(B)], axis=0)
    hnew = _elu(_mmT(hcat, fcw[l]) + fcbr[l]) + res_t
    mu = jnp.mean(hnew, axis=-1, keepdims=True)
    var = jnp.mean((hnew - mu) ** 2, axis=-1, keepdims=True)
    hln = (hnew - mu) / jnp.sqrt(var + 1e-5) * lng[...] + lnb[...]
    zpad2 = jnp.zeros((NP - NT, F), dtype=F32)
    hf = jnp.concatenate(
        sum([[hln[b * NT:(b + 1) * NT], zpad2] for b in range(B)], []),
        axis=0)

    l = 1
    h97 = jnp.concatenate(
        [hf[b * NP + LAST:b * NP + LAST + 1] for b in range(B)], axis=0)
    res97 = h97
    E2_l = []
    u2_l = []
    Wh2_l = []
    for hd in range(H):
        W = headW[l, hd]
        Wh = _mmT(hf, W)
        Wh2_l.append(Wh)
        Wh97 = jnp.concatenate(
            [Wh[b * NP + LAST:b * NP + LAST + 1] for b in range(B)], axis=0)
        si97T = _mmT(Ai[l, hd], Wh97)
        smT = _mmT(Am[l, hd], meta_a) + eb1c[l, hd]
        tu = jnp.maximum(_mmT(hf, uw1[l, hd]) + ub1r[l, hd], 0.0)
        for b in range(B):
            Whb = Wh[b * NP:(b + 1) * NP]
            TJt = _mmT(Aj[l, hd], Whb) + smT[:, b:b + 1]
            Hrow = jnp.maximum(TJt + si97T[:, b:b + 1], 0.0)
            erow = jnp.sum(Hrow * ew2c[l, hd], axis=0, keepdims=True) \
                + eb2[l, hd]
            E2_l.append(erow)
            u2_l.append(_sigmoid(_mmT(uw2r[l, hd], tu[b * NP:(b + 1) * NP])
                                 + ub2[l, hd]))
    E2 = jnp.concatenate(E2_l, axis=0)
    att2 = _masked_softmax_att(E2, colI2)
    outs_b = [[] for _ in range(B)]
    for hd in range(H):
        for b in range(B):
            row = att2[hd * B + b:hd * B + b + 1] * u2_l[hd * B + b]
            outs_b[b].append(_mm(row, Wh2_l[hd][b * NP:(b + 1) * NP]))
    hcat2 = jnp.concatenate(
        [jnp.concatenate(outs_b[b], axis=1) for b in range(B)], axis=0)
    hn2 = _elu(_mmT(hcat2, fcw[l]) + fcbr[l]) + res97
    mu2 = jnp.mean(hn2, axis=-1, keepdims=True)
    var2 = jnp.mean((hn2 - mu2) ** 2, axis=-1, keepdims=True)
    h2 = (hn2 - mu2) / jnp.sqrt(var2 + 1e-5) * lng[...] + lnb[...]

    out[...] = _mmT(h2, projw[...]) + projbr[...]


def kernel(hs, meta, pool_fc1_w, pool_fc1_b, pool_fc2_w, pool_fc2_b,
           cls_token, head_W, edge_w1, edge_b1, edge_w2, edge_b2,
           unc_w1, unc_b1, unc_w2, unc_b2, fc_w, fc_b, ln_g, ln_b,
           proj_w, proj_b):
    Ai = edge_w1[:, :, :, 0:F]
    Aj = edge_w1[:, :, :, F:2 * F]
    Am = edge_w1[:, :, :, 2 * F:2 * F + F]
    hs8 = hs.reshape(2 * B, NS, F)
    scores = pl.pallas_call(
        _scores_body,
        out_shape=jax.ShapeDtypeStruct((2 * B, NS), F32),
    )(hs8, pool_fc1_w, pool_fc1_b.reshape(2, 1, F),
      pool_fc2_w.reshape(2, 1, F))
    pooled8 = _pool_sc(scores, hs8)
    args = (
        pooled8,
        meta,
        cls_token.reshape(1, F),
        head_W,
        Ai, Aj, Am,
        edge_b1.reshape(2, H, HE, 1),
        edge_w2.reshape(2, H, HE, 1),
        edge_b2.reshape(2, H, 1, 1),
        unc_w1,
        unc_b1.reshape(2, H, 1, HE),
        unc_w2.reshape(2, H, 1, HE),
        unc_b2.reshape(2, H, 1, 1),
        fc_w,
        fc_b.reshape(2, 1, F),
        ln_g.reshape(1, F),
        ln_b.reshape(1, F),
        proj_w,
        proj_b.reshape(1, F),
    )
    return pl.pallas_call(
        _body,
        out_shape=jax.ShapeDtypeStruct((B, F), F32),
    )(*args)


# speedup vs baseline: 1.4100x; 1.4100x over previous
import jax
import jax.numpy as jnp
from jax.experimental import pallas as pl

F32 = jnp.float32
NEG = -1e30
B = 4
NS = 128
F = 128
H = 4
HE = 64
KPOOL = (64, 32)
NREAL = 98
NP = 128
KEDGE = 10
LAST = 97
NT = 104

_PREC = jax.lax.Precision.DEFAULT


def _mm(a, b):
    return jax.lax.dot_general(a, b, (((1,), (0,)), ((), ())),
                               precision=_PREC, preferred_element_type=F32)


def _mmT(a, b):
    return jax.lax.dot_general(a, b, (((1,), (1,)), ((), ())),
                               precision=_PREC, preferred_element_type=F32)


def _sigmoid(x):
    return 1.0 / (1.0 + jnp.exp(-x))


def _elu(x):
    return jnp.where(x > 0, x, jnp.exp(jnp.minimum(x, 0.0)) - 1.0)


def _topk_mask(e, k):
    col = jax.lax.broadcasted_iota(jnp.int32, e.shape, e.ndim - 1)
    work = e
    mask = jnp.zeros_like(e)
    for _ in range(k):
        m = jnp.max(work, axis=-1, keepdims=True)
        cand = jnp.where(work == m, col, jnp.int32(1 << 20))
        first = jnp.min(cand, axis=-1, keepdims=True)
        sel = col == first
        mask = jnp.where(sel, 1.0, mask)
        work = jnp.where(sel, NEG, work)
    return mask


def _masked_softmax_att(eraw, colI):
    valid = colI < NREAL
    esel = jnp.where(valid, eraw, NEG)
    mask = _topk_mask(esel, KEDGE)
    e = jnp.where(valid, eraw * mask, NEG)
    m = jnp.max(e, axis=-1, keepdims=True)
    p = jnp.exp(e - m)
    return p / jnp.sum(p, axis=-1, keepdims=True)


def _body(hs, meta, pw1, pb1, pw2, cls, headW, Ai, Aj, Am, eb1c, ew2c, eb2,
          uw1, ub1r, uw2r, ub2, fcw, fcbr, lng, lnb, projw, projbr, out):
    meta_a = meta[...]
    cls_r = cls[...]

    pooled = [[None] * B, [None] * B]
    for s in range(2):
        w1 = pw1[s]
        b1 = pb1[s]
        w2 = pw2[s]
        for b in range(B):
            X = hs[s, b]
            t1 = jnp.tanh(_mmT(X, w1) + b1)
            sc_row = _mmT(w2, t1)
            sc_col = jnp.transpose(sc_row, (1, 0))
            iom = jax.lax.broadcasted_iota(jnp.int32, (NS, NS), 0)
            ion = jax.lax.broadcasted_iota(jnp.int32, (NS, NS), 1)
            better = (sc_col > sc_row) | ((sc_col == sc_row) & (iom < ion))
            rank = jnp.sum(better.astype(F32), axis=0, keepdims=True)
            K = KPOOL[s]
            tI = jax.lax.broadcasted_iota(jnp.int32, (K, NS), 0).astype(F32)
            P = (tI == rank).astype(F32)
            pooled[s][b] = _mm(P, X)

    zpad = jnp.zeros((NP - NREAL, F), dtype=F32)
    hrows = []
    for b in range(B):
        hrows.append(jnp.concatenate(
            [pooled[0][b], pooled[1][b], meta_a[b:b + 1, :], cls_r, zpad],
            axis=0))
    hf = jnp.concatenate(hrows, axis=0)

    colI = jax.lax.broadcasted_iota(jnp.int32, (H * B * NT, NP), 1)
    colI2 = jax.lax.broadcasted_iota(jnp.int32, (H * B, NP), 1)

    l = 0
    res = hf
    Wh_l = []
    u_l = []
    E_l = []
    for hd in range(H):
        W = headW[l, hd]
        Wh = _mmT(hf, W)
        Wh_l.append(Wh)
        si = _mmT(Wh, Ai[l, hd])
        smT = _mmT(Am[l, hd], meta_a) + eb1c[l, hd]
        tu = jnp.maximum(_mmT(hf, uw1[l, hd]) + ub1r[l, hd], 0.0)
        ub = []
        for b in range(B):
            Whb = Wh[b * NP:(b + 1) * NP]
            TJt = _mmT(Aj[l, hd], Whb) + smT[:, b:b + 1]
            sib = si[b * NP:b * NP + NT]
            Hd = jnp.maximum(sib[:, :, None] + TJt[None, :, :], 0.0)
            Eb = jnp.sum(Hd * ew2c[l, hd][None, :, :], axis=1) + eb2[l, hd]
            E_l.append(Eb)
            utb = _sigmoid(_mmT(uw2r[l, hd], tu[b * NP:(b + 1) * NP])
                           + ub2[l, hd])
            ub.append(utb)
        u_l.append(ub)

    E_all = jnp.concatenate(E_l, axis=0)
    att_all = _masked_softmax_att(E_all, colI)
    hcat_b = [[] for _ in range(B)]
    for hd in range(H):
        for b in range(B):
            att = att_all[(hd * B + b) * NT:(hd * B + b + 1) * NT]
            att = att * u_l[hd][b]
            hcat_b[b].append(_mm(att, Wh_l[hd][b * NP:(b + 1) * NP]))
    hcat = jnp.concatenate(
        [jnp.concatenate(hcat_b[b], axis=1) for b in range(B)], axis=0)
    res_t = jnp.concatenate(
        [hf[b * NP:b * NP + NT] for b in range(B)], axis=0)
    hnew = _elu(_mmT(hcat, fcw[l]) + fcbr[l]) + res_t
    mu = jnp.mean(hnew, axis=-1, keepdims=True)
    var = jnp.mean((hnew - mu) ** 2, axis=-1, keepdims=True)
    hln = (hnew - mu) / jnp.sqrt(var + 1e-5) * lng[...] + lnb[...]
    zpad2 = jnp.zeros((NP - NT, F), dtype=F32)
    hf = jnp.concatenate(
        sum([[hln[b * NT:(b + 1) * NT], zpad2] for b in range(B)], []),
        axis=0)

    l = 1
    h97 = jnp.concatenate(
        [hf[b * NP + LAST:b * NP + LAST + 1] for b in range(B)], axis=0)
    res97 = h97
    E2_l = []
    u2_l = []
    Wh2_l = []
    for hd in range(H):
        W = headW[l, hd]
        Wh = _mmT(hf, W)
        Wh2_l.append(Wh)
        Wh97 = jnp.concatenate(
            [Wh[b * NP + LAST:b * NP + LAST + 1] for b in range(B)], axis=0)
        si97T = _mmT(Ai[l, hd], Wh97)
        smT = _mmT(Am[l, hd], meta_a) + eb1c[l, hd]
        tu = jnp.maximum(_mmT(hf, uw1[l, hd]) + ub1r[l, hd], 0.0)
        for b in range(B):
            Whb = Wh[b * NP:(b + 1) * NP]
            TJt = _mmT(Aj[l, hd], Whb) + smT[:, b:b + 1]
            Hrow = jnp.maximum(TJt + si97T[:, b:b + 1], 0.0)
            erow = jnp.sum(Hrow * ew2c[l, hd], axis=0, keepdims=True) \
                + eb2[l, hd]
            E2_l.append(erow)
            u2_l.append(_sigmoid(_mmT(uw2r[l, hd], tu[b * NP:(b + 1) * NP])
                                 + ub2[l, hd]))
    E2 = jnp.concatenate(E2_l, axis=0)
    att2 = _masked_softmax_att(E2, colI2)
    outs_b = [[] for _ in range(B)]
    for hd in range(H):
        for b in range(B):
            row = att2[hd * B + b:hd * B + b + 1] * u2_l[hd * B + b]
            outs_b[b].append(_mm(row, Wh2_l[hd][b * NP:(b + 1) * NP]))
    hcat2 = jnp.concatenate(
        [jnp.concatenate(outs_b[b], axis=1) for b in range(B)], axis=0)
    hn2 = _elu(_mmT(hcat2, fcw[l]) + fcbr[l]) + res97
    mu2 = jnp.mean(hn2, axis=-1, keepdims=True)
    var2 = jnp.mean((hn2 - mu2) ** 2, axis=-1, keepdims=True)
    h2 = (hn2 - mu2) / jnp.sqrt(var2 + 1e-5) * lng[...] + lnb[...]

    out[...] = _mmT(h2, projw[...]) + projbr[...]


def kernel(hs, meta, pool_fc1_w, pool_fc1_b, pool_fc2_w, pool_fc2_b,
           cls_token, head_W, edge_w1, edge_b1, edge_w2, edge_b2,
           unc_w1, unc_b1, unc_w2, unc_b2, fc_w, fc_b, ln_g, ln_b,
           proj_w, proj_b):
    Ai = edge_w1[:, :, :, 0:F]
    Aj = edge_w1[:, :, :, F:2 * F]
    Am = edge_w1[:, :, :, 2 * F:2 * F + F]
    args = (
        hs,
        meta,
        pool_fc1_w,
        pool_fc1_b.reshape(2, 1, F),
        pool_fc2_w.reshape(2, 1, F),
        cls_token.reshape(1, F),
        head_W,
        Ai, Aj, Am,
        edge_b1.reshape(2, H, HE, 1),
        edge_w2.reshape(2, H, HE, 1),
        edge_b2.reshape(2, H, 1, 1),
        unc_w1,
        unc_b1.reshape(2, H, 1, HE),
        unc_w2.reshape(2, H, 1, HE),
        unc_b2.reshape(2, H, 1, 1),
        fc_w,
        fc_b.reshape(2, 1, F),
        ln_g.reshape(1, F),
        ln_b.reshape(1, F),
        proj_w,
        proj_b.reshape(1, F),
    )
    return pl.pallas_call(
        _body,
        out_shape=jax.ShapeDtypeStruct((B, F), F32),
    )(*args)
